# speedup vs baseline: 8.1221x; 8.1221x over previous
import functools

import jax
import jax.numpy as jnp
from jax.experimental import pallas as pl

_HIDDEN = 512
_SUBSEQ = 200
_K = 5
_SORT_W = 1024

_MLP_BLK = 512
_RANK_BLK = 256
_MERGE_BLK = 512
_KNN_BLK = 256
_SORT_BLK = 256


def _mlp_kernel(z_ref, w1_ref, b1_ref, w2_ref, b2_ref, out_ref):
    h = jnp.dot(z_ref[...], w1_ref[...], preferred_element_type=jnp.float32)
    h = jnp.maximum(h + b1_ref[...], 0.0)
    out_ref[...] = jnp.dot(h, w2_ref[...], preferred_element_type=jnp.float32) + b2_ref[...]


def _zsim_kernel(n, blk, zp_ref, zr_ref, out_ref):
    i = pl.program_id(0)
    s = jnp.sum(zp_ref[...] * zr_ref[...], axis=1, keepdims=True)
    rows = i * blk + jax.lax.broadcasted_iota(jnp.int32, (blk, 1), 0)
    out_ref[...] = jnp.where(rows == n - 1, -100000000.0, s)


def _rank_kernel(n, blk, comp_count, scol_ref, srow_ref, out_ref):
    i = pl.program_id(0)
    sc = scol_ref[...]
    sr = srow_ref[...]
    jidx = jax.lax.broadcasted_iota(jnp.int32, (1, n), 1)
    iidx = i * blk + jax.lax.broadcasted_iota(jnp.int32, (blk, 1), 0)
    lt = (sr < sc).astype(jnp.float32)
    eqlt = ((sr == sc) & (jidx < iidx)).astype(jnp.float32)
    cnt = jnp.sum(lt + eqlt, axis=1, keepdims=True)
    out_ref[...] = (cnt < float(comp_count)).astype(jnp.float32)


def _merge_kernel(z_ref, zr_ref, y_ref, yr_ref, comp_ref, zn_ref, ym_ref):
    comp = comp_ref[...] > 0.5
    z = z_ref[...]
    zm = jnp.where(comp, z, (z + zr_ref[...]) / 2.0)
    nrm = jnp.sqrt(jnp.sum(zm * zm, axis=1, keepdims=True))
    zn_ref[...] = zm / jnp.maximum(nrm, 1e-12)
    y = y_ref[...]
    ym_ref[...] = jnp.where(comp, y, jnp.minimum(y, yr_ref[...]))


def _knn_kernel(n, blk, zi_ref, zall_ref, vals_ref, idxs_ref):
    a = jax.lax.dot_general(
        zi_ref[...], zall_ref[...], (((1,), (1,)), ((), ())),
        preferred_element_type=jnp.float32)
    lane = jax.lax.broadcasted_iota(jnp.int32, (blk, n), 1)
    vals, idxs = [], []
    for _ in range(_K + 1):
        m = jnp.max(a, axis=1, keepdims=True)
        am = jnp.min(jnp.where(a == m, lane, n), axis=1, keepdims=True)
        vals.append(m)
        idxs.append(am)
        a = jnp.where(lane == am, -jnp.inf, a)
    vals_ref[...] = jnp.concatenate(vals, axis=1)
    idxs_ref[...] = jnp.concatenate(idxs, axis=1)


def _partner(x, d, low):
    up = jnp.concatenate([x[:, d:], x[:, :d]], axis=1)
    dn = jnp.concatenate([x[:, -d:], x[:, :-d]], axis=1)
    return jnp.where(low, up, dn)


def _sort_kernel(l, blk, x_ref, xr_ref, m_ref, mr_ref, comp_ref, xm_ref, mm_ref):
    lane = jax.lax.broadcasted_iota(jnp.int32, (blk, _SORT_W), 1)
    comp = comp_ref[...] > 0.5
    pad = _SORT_W - 2 * l
    key = jnp.concatenate(
        [m_ref[...], mr_ref[...], jnp.full((blk, pad), -1.0, jnp.float32)], axis=1)
    zero_zone = comp & (lane >= 2 * l - _SUBSEQ) & (lane < 2 * l)
    key = jnp.where(zero_zone, 0.0, key)
    pay = jnp.concatenate(
        [x_ref[...], xr_ref[...], jnp.zeros((blk, pad), jnp.float32)], axis=1)
    idx = lane

    kk = 2
    while kk <= _SORT_W:
        desc = (lane & kk) == 0
        d = kk // 2
        while d >= 1:
            low = (lane & d) == 0
            pk = _partner(key, d, low)
            pi = _partner(idx, d, low)
            pp = _partner(pay, d, low)
            gt = (key > pk) | ((key == pk) & (idx < pi))
            take_self = gt == (desc == low)
            key = jnp.where(take_self, key, pk)
            idx = jnp.where(take_self, idx, pi)
            pay = jnp.where(take_self, pay, pp)
            d //= 2
        kk *= 2

    xm_ref[...] = pay[:, :2 * l]
    mm_ref[...] = key[:, :2 * l]


def kernel(z, x, y, x_mask, temporal_edge_index, temporal_edge_attr,
           sliding_wdw, W1, b1, W2, b2):
    n, dd = z.shape
    l = x.shape[1]
    merge_num = n // 2
    f32 = jnp.float32

    z_proj = pl.pallas_call(
        _mlp_kernel,
        grid=(n // _MLP_BLK,),
        in_specs=[
            pl.BlockSpec((_MLP_BLK, dd), lambda i: (i, 0)),
            pl.BlockSpec((dd, _HIDDEN), lambda i: (0, 0)),
            pl.BlockSpec((1, _HIDDEN), lambda i: (0, 0)),
            pl.BlockSpec((_HIDDEN, _HIDDEN), lambda i: (0, 0)),
            pl.BlockSpec((1, _HIDDEN), lambda i: (0, 0)),
        ],
        out_specs=pl.BlockSpec((_MLP_BLK, _HIDDEN), lambda i: (i, 0)),
        out_shape=jax.ShapeDtypeStruct((n, _HIDDEN), f32),
    )(z, W1, b1.reshape(1, -1), W2, b2.reshape(1, -1))

    zp_roll = jnp.roll(z_proj, -1, axis=0)
    z_sim = pl.pallas_call(
        functools.partial(_zsim_kernel, n, _MLP_BLK),
        grid=(n // _MLP_BLK,),
        in_specs=[
            pl.BlockSpec((_MLP_BLK, _HIDDEN), lambda i: (i, 0)),
            pl.BlockSpec((_MLP_BLK, _HIDDEN), lambda i: (i, 0)),
        ],
        out_specs=pl.BlockSpec((_MLP_BLK, 1), lambda i: (i, 0)),
        out_shape=jax.ShapeDtypeStruct((n, 1), f32),
    )(z_proj, zp_roll)

    comp = pl.pallas_call(
        functools.partial(_rank_kernel, n, _RANK_BLK, n - merge_num),
        grid=(n // _RANK_BLK,),
        in_specs=[
            pl.BlockSpec((_RANK_BLK, 1), lambda i: (i, 0)),
            pl.BlockSpec((1, n), lambda i: (0, 0)),
        ],
        out_specs=pl.BlockSpec((_RANK_BLK, 1), lambda i: (i, 0)),
        out_shape=jax.ShapeDtypeStruct((n, 1), f32),
    )(z_sim, z_sim.reshape(1, n))

    z_roll = jnp.roll(z, -1, axis=0)
    y2 = y.reshape(n, 1)
    y2_roll = jnp.roll(y2, -1, axis=0)
    z_n, y_merge2 = pl.pallas_call(
        _merge_kernel,
        grid=(n // _MERGE_BLK,),
        in_specs=[
            pl.BlockSpec((_MERGE_BLK, dd), lambda i: (i, 0)),
            pl.BlockSpec((_MERGE_BLK, dd), lambda i: (i, 0)),
            pl.BlockSpec((_MERGE_BLK, 1), lambda i: (i, 0)),
            pl.BlockSpec((_MERGE_BLK, 1), lambda i: (i, 0)),
            pl.BlockSpec((_MERGE_BLK, 1), lambda i: (i, 0)),
        ],
        out_specs=[
            pl.BlockSpec((_MERGE_BLK, dd), lambda i: (i, 0)),
            pl.BlockSpec((_MERGE_BLK, 1), lambda i: (i, 0)),
        ],
        out_shape=[
            jax.ShapeDtypeStruct((n, dd), f32),
            jax.ShapeDtypeStruct((n, 1), f32),
        ],
    )(z, z_roll, y2, y2_roll, comp)

    vals, idxs = pl.pallas_call(
        functools.partial(_knn_kernel, n, _KNN_BLK),
        grid=(n // _KNN_BLK,),
        in_specs=[
            pl.BlockSpec((_KNN_BLK, dd), lambda i: (i, 0)),
            pl.BlockSpec((n, dd), lambda i: (0, 0)),
        ],
        out_specs=[
            pl.BlockSpec((_KNN_BLK, _K + 1), lambda i: (i, 0)),
            pl.BlockSpec((_KNN_BLK, _K + 1), lambda i: (i, 0)),
        ],
        out_shape=[
            jax.ShapeDtypeStruct((n, _K + 1), f32),
            jax.ShapeDtypeStruct((n, _K + 1), jnp.int32),
        ],
    )(z_n, z_n)

    x_roll = jnp.roll(x, -1, axis=0)
    m_roll = jnp.roll(x_mask, -1, axis=0)
    x_merge, x_mask_merge = pl.pallas_call(
        functools.partial(_sort_kernel, l, _SORT_BLK),
        grid=(n // _SORT_BLK,),
        in_specs=[
            pl.BlockSpec((_SORT_BLK, l), lambda i: (i, 0)),
            pl.BlockSpec((_SORT_BLK, l), lambda i: (i, 0)),
            pl.BlockSpec((_SORT_BLK, l), lambda i: (i, 0)),
            pl.BlockSpec((_SORT_BLK, l), lambda i: (i, 0)),
            pl.BlockSpec((_SORT_BLK, 1), lambda i: (i, 0)),
        ],
        out_specs=[
            pl.BlockSpec((_SORT_BLK, 2 * l), lambda i: (i, 0)),
            pl.BlockSpec((_SORT_BLK, 2 * l), lambda i: (i, 0)),
        ],
        out_shape=[
            jax.ShapeDtypeStruct((n, 2 * l), f32),
            jax.ShapeDtypeStruct((n, 2 * l), f32),
        ],
    )(x, x_roll, x_mask, m_roll, comp)

    dist = vals[:, 1:]
    idx = idxs[:, 1:]
    idx_source = jnp.repeat(jnp.arange(n, dtype=jnp.int32), _K)
    edge_index = jnp.stack([idx_source, idx.reshape(-1)], axis=0)
    attr = dist.reshape(-1, 1)
    return (x_merge, edge_index, attr, y_merge2.reshape(n),
            temporal_edge_index, temporal_edge_attr, x_mask_merge)
